# baseline (speedup 1.0000x reference)
import functools

import jax
import jax.numpy as jnp
from jax import lax
from jax.experimental import pallas as pl
from jax.experimental.pallas import tpu as pltpu
from jax.experimental.pallas import tpu_sc as plsc

_N = 10000
_E = 320000
_H = 4
_OC = 16
_HOC = _H * _OC
_NC = 2
_NS = 16
_NW = _NC * _NS
_L = 16
_EPW = _E // _NW
_BE = 80
_NCH = _EPW // _BE
_RPT = 632
_ROW_LAST = _N - _RPT

_sc_mesh = plsc.VectorSubcoreMesh(core_axis_name="c", subcore_axis_name="s")


def _make_edge_kernel(dp):
    @functools.partial(
        pl.kernel,
        out_type=jax.ShapeDtypeStruct((_NC * _N, dp), jnp.float32),
        mesh=_sc_mesh,
        scratch_types=[
            pltpu.VMEM_SHARED((_N, dp), jnp.float32),
            pltpu.VMEM((3, _BE), jnp.int32),
            pltpu.VMEM((3, _BE), jnp.int32),
            pltpu.VMEM((_N * _H,), jnp.float32),
            pltpu.VMEM((_L,), jnp.float32),
            pltpu.VMEM((3, _BE, _HOC), jnp.float32),
            pltpu.VMEM((3, _BE, dp), jnp.float32),
            pltpu.VMEM((3, _H + 1, _BE), jnp.float32),
            pltpu.VMEM((3, _BE), jnp.int32),
            pltpu.VMEM((3, _BE), jnp.int32),
            pltpu.VMEM((3, _BE), jnp.int32),
        ] + [pltpu.SemaphoreType.DMA] * 9,
        compiler_params=pltpu.CompilerParams(needs_layout_passes=False,
                                             use_tc_tiling_on_sc=False),
    )
    def edge_kernel(src_hbm, dst_hbm, xw_hbm, xu_hbm, c_hbm, out_hbm,
                    agg, src_v, dst_v, xu_v, c_v, rows_v, msg_v, q_v,
                    sidx_v, didx_v, dstage_v, sem0, sem1, sem2, ssem0, ssem1,
                    ssem2, esem0, esem1, esem2):
        ci = lax.axis_index("c")
        si = lax.axis_index("s")
        w = ci * _NS + si
        sems = (sem0, sem1, sem2)
        ssems = (ssem0, ssem1, ssem2)
        esems = (esem0, esem1, esem2)
        zeroi = jnp.zeros((_L,), jnp.int32)

        zero16 = jnp.zeros((_L,), jnp.float32)
        for s in range(3):
            for g in range(_BE // _L):
                didx_v[s, pl.ds(g * _L, _L)] = zeroi
            for r in range(_BE):
                for col in range(dp // _L):
                    msg_v[s, r, pl.ds(col * _L, _L)] = zero16
                if dp % _L:
                    msg_v[s, r, pl.ds(dp - _L, _L)] = zero16

        row0 = jnp.minimum(si * _RPT, _ROW_LAST)
        for k in range(_RPT // _BE):
            pltpu.sync_copy(msg_v.at[0], agg.at[pl.ds(row0 + k * _BE, _BE)])
        _rem = _RPT - (_RPT // _BE) * _BE
        if _rem:
            pltpu.sync_copy(msg_v.at[0, pl.ds(0, _rem)],
                            agg.at[pl.ds(row0 + _RPT - _rem, _rem)])

        pltpu.sync_copy(xu_hbm, xu_v)
        pltpu.sync_copy(c_hbm, c_v)
        cvec = c_v[...]
        cs = [cvec[h] for h in range(_H)]

        plsc.subcore_barrier()

        for s in range(3):
            pltpu.async_copy(msg_v.at[s], agg.at[didx_v.at[s]], ssems[s],
                             add=True)

        base_e = w * _EPW

        def stage(k, slot):
            off = base_e + jnp.minimum(k, _NCH - 1) * _BE
            pltpu.async_copy(src_hbm.at[pl.ds(off, _BE)], src_v.at[slot],
                             esems[slot])
            pltpu.async_copy(dst_hbm.at[pl.ds(off, _BE)], dst_v.at[slot],
                             esems[slot])

        def stage_wait(slot):
            for _ in range(2):
                pltpu.make_async_copy(src_hbm.at[pl.ds(base_e, _BE)],
                                      src_v.at[slot], esems[slot]).wait()

        def phase_a(k, slot):
            stage_wait(slot)
            for g in range(_BE // _L):
                sg = src_v[slot, pl.ds(g * _L, _L)]
                dg = dst_v[slot, pl.ds(g * _L, _L)]
                dstage_v[slot, pl.ds(g * _L, _L)] = dg
                sidx_v[slot, pl.ds(g * _L, _L)] = sg
                sg4 = sg * _H
                dg4 = dg * _H
                xus = [plsc.load_gather(xu_v, [sg4 + h if h else sg4])
                       for h in range(_H)]
                xud = [plsc.load_gather(xu_v, [dg4 + h if h else dg4])
                       for h in range(_H)]
                ts = [xus[h] - xud[h] + cs[h] for h in range(_H)]
                es = [jnp.exp(ts[h]) for h in range(_H)]
                ssum = (es[0] + es[1]) + (es[2] + es[3])
                validf = jnp.where(sg != dg, 1.0, 0.0).astype(jnp.float32)
                scale = validf / ssum
                for h in range(_H):
                    q_v[slot, h, pl.ds(g * _L, _L)] = es[h] * scale
                q_v[slot, _H, pl.ds(g * _L, _L)] = validf
            pltpu.async_copy(xw_hbm.at[sidx_v.at[slot]],
                             rows_v.at[slot], sems[slot])

        def phase_b(slot):
            pltpu.make_async_copy(msg_v.at[slot], agg.at[didx_v.at[slot]],
                                  ssems[slot]).wait()
            for g in range(_BE // _L):
                dg = dstage_v[slot, pl.ds(g * _L, _L)]
                didx_v[slot, pl.ds(g * _L, _L)] = dg
                if dp > _OC:
                    validf = q_v[slot, _H, pl.ds(g * _L, _L)]
                    ridx = lax.iota(jnp.int32, _L) + (g * _L)
                    cidx = jnp.full((_L,), _OC, jnp.int32)
                    plsc.store_scatter(msg_v.at[slot], [ridx, cidx], validf)
            pltpu.make_async_copy(xw_hbm.at[sidx_v.at[slot]],
                                  rows_v.at[slot], sems[slot]).wait()
            for g in range(_BE // _L):
                qv = [q_v[slot, h, pl.ds(g * _L, _L)] for h in range(_H)]
                for lane in range(_L):
                    e = g * _L + lane
                    r0 = rows_v[slot, e, pl.ds(0, _L)]
                    r1 = rows_v[slot, e, pl.ds(_L, _L)]
                    r2 = rows_v[slot, e, pl.ds(2 * _L, _L)]
                    r3 = rows_v[slot, e, pl.ds(3 * _L, _L)]
                    msg_v[slot, e, pl.ds(0, _L)] = (
                        (qv[0][lane] * r0 + qv[1][lane] * r1)
                        + (qv[2][lane] * r2 + qv[3][lane] * r3))
            pltpu.async_copy(msg_v.at[slot], agg.at[didx_v.at[slot]],
                             ssems[slot], add=True)

        for s in range(3):
            stage(s, s)
        for s in range(3):
            phase_a(s, s)
            stage(s + 3, s)

        def trip_body(k3, c3):
            k = 3 * k3
            for s in range(3):
                phase_b(s)
                phase_a(k + s + 3, s)
                stage(k + s + 6, s)
            return c3

        lax.fori_loop(0, (_NCH - 2) // 3, trip_body, 0)
        phase_b(0)
        phase_b(1)
        for s in range(3):
            stage_wait(s)
            pltpu.make_async_copy(msg_v.at[s], agg.at[didx_v.at[s]],
                                  ssems[s]).wait()
        pltpu.make_async_copy(xw_hbm.at[sidx_v.at[2]],
                              rows_v.at[2], sems[2]).wait()
        plsc.subcore_barrier()
        pltpu.sync_copy(agg.at[pl.ds(row0, _RPT)],
                        out_hbm.at[pl.ds(ci * _N + row0, _RPT)])

    return edge_kernel


_DPA = 24
_edge_kernel_a = _make_edge_kernel(_DPA)
_edge_kernel_b = _make_edge_kernel(_OC)


def _tc_pre_body(h_ref, W_ref, u_ref, wself_ref, xw_ref, xu_ref, sm_ref):
    h = h_ref[...]
    xw_ref[...] = jnp.dot(h, W_ref[...], preferred_element_type=jnp.float32)
    xu_ref[...] = jnp.dot(h, u_ref[...], preferred_element_type=jnp.float32)
    sm_ref[...] = jnp.dot(h, wself_ref[...], preferred_element_type=jnp.float32)


def _tc_mid1_body(parts_ref, sm_ref, b_ref, dsel_ref, W_ref, u_ref, wself_ref,
                  xw_ref, xu_ref, smn_ref, invdeg_ref):
    s = parts_ref[0:_N, :] + parts_ref[_N:2 * _N, :]
    msg = s[:, 0:_OC]
    deg = jnp.dot(s, dsel_ref[...], preferred_element_type=jnp.float32) + 1.0
    invdeg = 1.0 / deg
    h = jnp.maximum(0.0, (msg + sm_ref[...]) * invdeg + b_ref[...])
    invdeg_ref[...] = invdeg
    xw_ref[...] = jnp.dot(h, W_ref[...], preferred_element_type=jnp.float32)
    xu_ref[...] = jnp.dot(h, u_ref[...], preferred_element_type=jnp.float32)
    smn_ref[...] = jnp.dot(h, wself_ref[...],
                           preferred_element_type=jnp.float32)


def _combine16(parts, sm, b, invdeg):
    s = parts[0:_N, :] + parts[_N:2 * _N, :]
    return jnp.maximum(0.0, (s + sm) * invdeg + b)


def _tc_mid2_body(parts_ref, sm_ref, b_ref, invdeg_ref, W_ref, u_ref,
                  wself_ref, xw_ref, xu_ref, smn_ref):
    h = _combine16(parts_ref[...], sm_ref[...], b_ref[...], invdeg_ref[...])
    xw_ref[...] = jnp.dot(h, W_ref[...], preferred_element_type=jnp.float32)
    xu_ref[...] = jnp.dot(h, u_ref[...], preferred_element_type=jnp.float32)
    smn_ref[...] = jnp.dot(h, wself_ref[...],
                           preferred_element_type=jnp.float32)


def _tc_head_body(parts_ref, sm_ref, b_ref, invdeg_ref, lw1_ref, lb1_ref,
                  lw2_ref, lb2_ref, ow_ref, ob_ref, out_ref, sig_ref):
    h = _combine16(parts_ref[...], sm_ref[...], b_ref[...], invdeg_ref[...])
    h1 = jnp.maximum(0.0, jnp.dot(h, lw1_ref[...],
                                  preferred_element_type=jnp.float32)
                     + lb1_ref[...])
    inter = jnp.dot(h1, lw2_ref[...],
                    preferred_element_type=jnp.float32) + lb2_ref[...]
    h2 = jnp.maximum(0.0, inter)
    out_ref[...] = jax.nn.sigmoid(
        jnp.dot(h2, ow_ref[...], preferred_element_type=jnp.float32)
        + ob_ref[...])
    sig_ref[...] = jax.nn.sigmoid(inter)


def _f32(shape):
    return jax.ShapeDtypeStruct(shape, jnp.float32)


_tc_pre = pl.pallas_call(
    _tc_pre_body,
    out_shape=[_f32((_N, _HOC)), _f32((_N, _H)), _f32((_N, _OC))],
)

_tc_mid1 = pl.pallas_call(
    _tc_mid1_body,
    out_shape=[_f32((_N, _HOC)), _f32((_N, _H)), _f32((_N, _OC)),
               _f32((_N, 1))],
)

_tc_mid2 = pl.pallas_call(
    _tc_mid2_body,
    out_shape=[_f32((_N, _HOC)), _f32((_N, _H)), _f32((_N, _OC))],
)

_tc_head = pl.pallas_call(
    _tc_head_body,
    out_shape=[_f32((_N, 1)), _f32((_N, _H))],
)


def kernel(x, edge_index, W1, u1, c1, b1, W2, u2, c2, b2, W3, u3, c3, b3,
           lw1, lb1, lw2, lb2, ow, ob):
    f32 = jnp.float32
    src = edge_index[0]
    dst = edge_index[1]

    def prep(W, c):
        qs = jax.nn.softmax(c)
        wself = (W.reshape(-1, _H, _OC) * qs[None, :, None]).sum(axis=1)
        cpad = jnp.concatenate([c, jnp.zeros((_L - _H,), f32)])
        return wself, cpad

    wself1, cp1 = prep(W1, c1)
    wself2, cp2 = prep(W2, c2)
    wself3, cp3 = prep(W3, c3)
    dsel = jnp.zeros((_DPA, 1), f32).at[_OC, 0].set(1.0)

    xw, xu, sm = _tc_pre(x, W1, u1, wself1)
    parts = _edge_kernel_a(src, dst, xw, xu.reshape(-1), cp1)
    xw, xu, sm, invdeg = _tc_mid1(parts, sm, b1.reshape(1, _OC), dsel,
                                  W2, u2, wself2)
    parts = _edge_kernel_b(src, dst, xw, xu.reshape(-1), cp2)
    xw, xu, sm = _tc_mid2(parts, sm, b2.reshape(1, _OC), invdeg,
                          W3, u3, wself3)
    parts = _edge_kernel_b(src, dst, xw, xu.reshape(-1), cp3)
    out, sig = _tc_head(parts, sm, b3.reshape(1, _OC), invdeg,
                        lw1, lb1.reshape(1, -1), lw2, lb2.reshape(1, -1),
                        ow, ob.reshape(1, -1))
    return (out, sig)


# speedup vs baseline: 1.1429x; 1.1429x over previous
import functools

import jax
import jax.numpy as jnp
from jax import lax
from jax.experimental import pallas as pl
from jax.experimental.pallas import tpu as pltpu
from jax.experimental.pallas import tpu_sc as plsc

_N = 10000
_E = 320000
_H = 4
_OC = 16
_HOC = _H * _OC
_NC = 2
_NS = 16
_NW = _NC * _NS
_L = 16
_EPW = _E // _NW
_BE = 80
_NCH = _EPW // _BE
_RPT = 632
_ROW_LAST = _N - _RPT

_sc_mesh = plsc.VectorSubcoreMesh(core_axis_name="c", subcore_axis_name="s")


def _make_edge_kernel(dp):
    @functools.partial(
        pl.kernel,
        out_type=jax.ShapeDtypeStruct((_NC * _N, dp), jnp.float32),
        mesh=_sc_mesh,
        scratch_types=[
            pltpu.VMEM_SHARED((_N, dp), jnp.float32),
            pltpu.VMEM((2, _BE), jnp.int32),
            pltpu.VMEM((2, _BE), jnp.int32),
            pltpu.VMEM((_N * _H,), jnp.float32),
            pltpu.VMEM((_L,), jnp.float32),
            pltpu.VMEM((2, _BE, _HOC), jnp.float32),
            pltpu.VMEM((2, _BE, dp), jnp.float32),
            pltpu.VMEM((2, _H + 1, _BE), jnp.float32),
            pltpu.VMEM((2, _BE), jnp.int32),
            pltpu.VMEM((2, _BE), jnp.int32),
            pltpu.VMEM((2, _BE), jnp.int32),
        ] + [pltpu.SemaphoreType.DMA] * 6,
        compiler_params=pltpu.CompilerParams(needs_layout_passes=False,
                                             use_tc_tiling_on_sc=False),
    )
    def edge_kernel(src_hbm, dst_hbm, xw_hbm, xu_hbm, c_hbm, out_hbm,
                    agg, src_v, dst_v, xu_v, c_v, rows_v, msg_v, q_v,
                    sidx_v, didx_v, dstage_v, sem0, sem1, ssem0, ssem1,
                    esem0, esem1):
        ci = lax.axis_index("c")
        si = lax.axis_index("s")
        w = ci * _NS + si
        sems = (sem0, sem1)
        ssems = (ssem0, ssem1)
        esems = (esem0, esem1)
        zeroi = jnp.zeros((_L,), jnp.int32)

        zero16 = jnp.zeros((_L,), jnp.float32)
        for s in range(2):
            for g in range(_BE // _L):
                didx_v[s, pl.ds(g * _L, _L)] = zeroi
            for r in range(_BE):
                for col in range(dp // _L):
                    msg_v[s, r, pl.ds(col * _L, _L)] = zero16
                if dp % _L:
                    msg_v[s, r, pl.ds(dp - _L, _L)] = zero16

        row0 = jnp.minimum(si * _RPT, _ROW_LAST)
        for k in range(_RPT // _BE):
            pltpu.sync_copy(msg_v.at[0], agg.at[pl.ds(row0 + k * _BE, _BE)])
        _rem = _RPT - (_RPT // _BE) * _BE
        if _rem:
            pltpu.sync_copy(msg_v.at[0, pl.ds(0, _rem)],
                            agg.at[pl.ds(row0 + _RPT - _rem, _rem)])

        pltpu.sync_copy(xu_hbm, xu_v)
        pltpu.sync_copy(c_hbm, c_v)
        cvec = c_v[...]
        cs = [cvec[h] for h in range(_H)]

        plsc.subcore_barrier()

        for s in range(2):
            pltpu.async_copy(msg_v.at[s], agg.at[didx_v.at[s]], ssems[s],
                             add=True)

        base_e = w * _EPW

        def stage(k, slot):
            off = base_e + jnp.minimum(k, _NCH - 1) * _BE
            pltpu.async_copy(src_hbm.at[pl.ds(off, _BE)], src_v.at[slot],
                             esems[slot])
            pltpu.async_copy(dst_hbm.at[pl.ds(off, _BE)], dst_v.at[slot],
                             esems[slot])

        def stage_wait(slot):
            for _ in range(2):
                pltpu.make_async_copy(src_hbm.at[pl.ds(base_e, _BE)],
                                      src_v.at[slot], esems[slot]).wait()

        def phase_a(k, slot):
            stage_wait(slot)
            for g in range(_BE // _L):
                sg = src_v[slot, pl.ds(g * _L, _L)]
                dg = dst_v[slot, pl.ds(g * _L, _L)]
                dstage_v[slot, pl.ds(g * _L, _L)] = dg
                sidx_v[slot, pl.ds(g * _L, _L)] = sg
                sg4 = sg * _H
                dg4 = dg * _H
                xus = [plsc.load_gather(xu_v, [sg4 + h if h else sg4])
                       for h in range(_H)]
                xud = [plsc.load_gather(xu_v, [dg4 + h if h else dg4])
                       for h in range(_H)]
                ts = [xus[h] - xud[h] + cs[h] for h in range(_H)]
                es = [jnp.exp(ts[h]) for h in range(_H)]
                ssum = (es[0] + es[1]) + (es[2] + es[3])
                validf = jnp.where(sg != dg, 1.0, 0.0).astype(jnp.float32)
                scale = validf / ssum
                for h in range(_H):
                    q_v[slot, h, pl.ds(g * _L, _L)] = es[h] * scale
                q_v[slot, _H, pl.ds(g * _L, _L)] = validf
            pltpu.async_copy(xw_hbm.at[sidx_v.at[slot]],
                             rows_v.at[slot], sems[slot])

        def phase_b(slot):
            pltpu.make_async_copy(msg_v.at[slot], agg.at[didx_v.at[slot]],
                                  ssems[slot]).wait()
            for g in range(_BE // _L):
                dg = dstage_v[slot, pl.ds(g * _L, _L)]
                didx_v[slot, pl.ds(g * _L, _L)] = dg
                if dp > _OC:
                    validf = q_v[slot, _H, pl.ds(g * _L, _L)]
                    ridx = lax.iota(jnp.int32, _L) + (g * _L)
                    cidx = jnp.full((_L,), _OC, jnp.int32)
                    plsc.store_scatter(msg_v.at[slot], [ridx, cidx], validf)
            pltpu.make_async_copy(xw_hbm.at[sidx_v.at[slot]],
                                  rows_v.at[slot], sems[slot]).wait()
            for g in range(_BE // _L):
                qv = [q_v[slot, h, pl.ds(g * _L, _L)] for h in range(_H)]
                for lane in range(_L):
                    e = g * _L + lane
                    r0 = rows_v[slot, e, pl.ds(0, _L)]
                    r1 = rows_v[slot, e, pl.ds(_L, _L)]
                    r2 = rows_v[slot, e, pl.ds(2 * _L, _L)]
                    r3 = rows_v[slot, e, pl.ds(3 * _L, _L)]
                    msg_v[slot, e, pl.ds(0, _L)] = (
                        (qv[0][lane] * r0 + qv[1][lane] * r1)
                        + (qv[2][lane] * r2 + qv[3][lane] * r3))
            pltpu.async_copy(msg_v.at[slot], agg.at[didx_v.at[slot]],
                             ssems[slot], add=True)

        stage(0, 0)
        stage(1, 1)
        phase_a(0, 0)
        stage(2, 0)

        def pair_body(k2, c2):
            k = 2 * k2
            phase_a(k + 1, 1)
            stage(k + 3, 1)
            phase_b(0)
            phase_a(k + 2, 0)
            stage(k + 4, 0)
            phase_b(1)
            return c2

        lax.fori_loop(0, (_NCH - 1) // 2, pair_body, 0)
        phase_b(0)
        for s in range(2):
            stage_wait(s)
            pltpu.make_async_copy(msg_v.at[s], agg.at[didx_v.at[s]],
                                  ssems[s]).wait()
        plsc.subcore_barrier()
        pltpu.sync_copy(agg.at[pl.ds(row0, _RPT)],
                        out_hbm.at[pl.ds(ci * _N + row0, _RPT)])

    return edge_kernel


_DPA = 24
_edge_kernel_a = _make_edge_kernel(_DPA)
_edge_kernel_b = _make_edge_kernel(_OC)


def _tc_pre_body(h_ref, W_ref, u_ref, wself_ref, xw_ref, xu_ref, sm_ref):
    h = h_ref[...]
    xw_ref[...] = jnp.dot(h, W_ref[...], preferred_element_type=jnp.float32)
    xu_ref[...] = jnp.dot(h, u_ref[...], preferred_element_type=jnp.float32)
    sm_ref[...] = jnp.dot(h, wself_ref[...], preferred_element_type=jnp.float32)


def _tc_mid1_body(parts_ref, sm_ref, b_ref, dsel_ref, W_ref, u_ref, wself_ref,
                  xw_ref, xu_ref, smn_ref, invdeg_ref):
    s = parts_ref[0:_N, :] + parts_ref[_N:2 * _N, :]
    msg = s[:, 0:_OC]
    deg = jnp.dot(s, dsel_ref[...], preferred_element_type=jnp.float32) + 1.0
    invdeg = 1.0 / deg
    h = jnp.maximum(0.0, (msg + sm_ref[...]) * invdeg + b_ref[...])
    invdeg_ref[...] = invdeg
    xw_ref[...] = jnp.dot(h, W_ref[...], preferred_element_type=jnp.float32)
    xu_ref[...] = jnp.dot(h, u_ref[...], preferred_element_type=jnp.float32)
    smn_ref[...] = jnp.dot(h, wself_ref[...],
                           preferred_element_type=jnp.float32)


def _combine16(parts, sm, b, invdeg):
    s = parts[0:_N, :] + parts[_N:2 * _N, :]
    return jnp.maximum(0.0, (s + sm) * invdeg + b)


def _tc_mid2_body(parts_ref, sm_ref, b_ref, invdeg_ref, W_ref, u_ref,
                  wself_ref, xw_ref, xu_ref, smn_ref):
    h = _combine16(parts_ref[...], sm_ref[...], b_ref[...], invdeg_ref[...])
    xw_ref[...] = jnp.dot(h, W_ref[...], preferred_element_type=jnp.float32)
    xu_ref[...] = jnp.dot(h, u_ref[...], preferred_element_type=jnp.float32)
    smn_ref[...] = jnp.dot(h, wself_ref[...],
                           preferred_element_type=jnp.float32)


def _tc_head_body(parts_ref, sm_ref, b_ref, invdeg_ref, lw1_ref, lb1_ref,
                  lw2_ref, lb2_ref, ow_ref, ob_ref, out_ref, sig_ref):
    h = _combine16(parts_ref[...], sm_ref[...], b_ref[...], invdeg_ref[...])
    h1 = jnp.maximum(0.0, jnp.dot(h, lw1_ref[...],
                                  preferred_element_type=jnp.float32)
                     + lb1_ref[...])
    inter = jnp.dot(h1, lw2_ref[...],
                    preferred_element_type=jnp.float32) + lb2_ref[...]
    h2 = jnp.maximum(0.0, inter)
    out_ref[...] = jax.nn.sigmoid(
        jnp.dot(h2, ow_ref[...], preferred_element_type=jnp.float32)
        + ob_ref[...])
    sig_ref[...] = jax.nn.sigmoid(inter)


def _f32(shape):
    return jax.ShapeDtypeStruct(shape, jnp.float32)


_tc_pre = pl.pallas_call(
    _tc_pre_body,
    out_shape=[_f32((_N, _HOC)), _f32((_N, _H)), _f32((_N, _OC))],
)

_tc_mid1 = pl.pallas_call(
    _tc_mid1_body,
    out_shape=[_f32((_N, _HOC)), _f32((_N, _H)), _f32((_N, _OC)),
               _f32((_N, 1))],
)

_tc_mid2 = pl.pallas_call(
    _tc_mid2_body,
    out_shape=[_f32((_N, _HOC)), _f32((_N, _H)), _f32((_N, _OC))],
)

_tc_head = pl.pallas_call(
    _tc_head_body,
    out_shape=[_f32((_N, 1)), _f32((_N, _H))],
)


def kernel(x, edge_index, W1, u1, c1, b1, W2, u2, c2, b2, W3, u3, c3, b3,
           lw1, lb1, lw2, lb2, ow, ob):
    f32 = jnp.float32
    src = edge_index[0]
    dst = edge_index[1]

    def prep(W, c):
        qs = jax.nn.softmax(c)
        wself = (W.reshape(-1, _H, _OC) * qs[None, :, None]).sum(axis=1)
        cpad = jnp.concatenate([c, jnp.zeros((_L - _H,), f32)])
        return wself, cpad

    wself1, cp1 = prep(W1, c1)
    wself2, cp2 = prep(W2, c2)
    wself3, cp3 = prep(W3, c3)
    dsel = jnp.zeros((_DPA, 1), f32).at[_OC, 0].set(1.0)

    xw, xu, sm = _tc_pre(x, W1, u1, wself1)
    parts = _edge_kernel_a(src, dst, xw, xu.reshape(-1), cp1)
    xw, xu, sm, invdeg = _tc_mid1(parts, sm, b1.reshape(1, _OC), dsel,
                                  W2, u2, wself2)
    parts = _edge_kernel_b(src, dst, xw, xu.reshape(-1), cp2)
    xw, xu, sm = _tc_mid2(parts, sm, b2.reshape(1, _OC), invdeg,
                          W3, u3, wself3)
    parts = _edge_kernel_b(src, dst, xw, xu.reshape(-1), cp3)
    out, sig = _tc_head(parts, sm, b3.reshape(1, _OC), invdeg,
                        lw1, lb1.reshape(1, -1), lw2, lb2.reshape(1, -1),
                        ow, ob.reshape(1, -1))
    return (out, sig)


# speedup vs baseline: 1.2170x; 1.0648x over previous
import functools

import jax
import jax.numpy as jnp
from jax import lax
from jax.experimental import pallas as pl
from jax.experimental.pallas import tpu as pltpu
from jax.experimental.pallas import tpu_sc as plsc

_N = 10000
_E = 320000
_H = 4
_OC = 16
_HOC = _H * _OC
_NC = 2
_NS = 16
_NW = _NC * _NS
_L = 16
_EPW = _E // _NW
_BE = 80
_NCH = _EPW // _BE
_RPT = 632
_ROW_LAST = _N - _RPT

_sc_mesh = plsc.VectorSubcoreMesh(core_axis_name="c", subcore_axis_name="s")


def _make_edge_kernel(dp):
    @functools.partial(
        pl.kernel,
        out_type=jax.ShapeDtypeStruct((_NC * _N, dp), jnp.float32),
        mesh=_sc_mesh,
        scratch_types=[
            pltpu.VMEM_SHARED((_N, dp), jnp.float32),
            pltpu.VMEM((2, _BE), jnp.int32),
            pltpu.VMEM((2, _BE), jnp.int32),
            pltpu.VMEM((_N * _H,), jnp.float32),
            pltpu.VMEM((_L,), jnp.float32),
            pltpu.VMEM((2, _BE, _HOC), jnp.float32),
            pltpu.VMEM((2, _BE, dp), jnp.float32),
            pltpu.VMEM((2, _H + 1, _BE), jnp.float32),
            pltpu.VMEM((2, _BE), jnp.int32),
            pltpu.VMEM((2, _BE), jnp.int32),
            pltpu.VMEM((2, _BE), jnp.int32),
        ] + [pltpu.SemaphoreType.DMA] * 6,
        compiler_params=pltpu.CompilerParams(needs_layout_passes=False,
                                             use_tc_tiling_on_sc=False),
    )
    def edge_kernel(src_hbm, dst_hbm, xw_hbm, xu_hbm, c_hbm, out_hbm,
                    agg, src_v, dst_v, xu_v, c_v, rows_v, msg_v, q_v,
                    sidx_v, didx_v, dstage_v, sem0, sem1, ssem0, ssem1,
                    esem0, esem1):
        ci = lax.axis_index("c")
        si = lax.axis_index("s")
        w = ci * _NS + si
        sems = (sem0, sem1)
        ssems = (ssem0, ssem1)
        esems = (esem0, esem1)
        zeroi = jnp.zeros((_L,), jnp.int32)

        zero16 = jnp.zeros((_L,), jnp.float32)
        for s in range(2):
            for g in range(_BE // _L):
                didx_v[s, pl.ds(g * _L, _L)] = zeroi
            for r in range(_BE):
                for col in range(dp // _L):
                    msg_v[s, r, pl.ds(col * _L, _L)] = zero16
                if dp % _L:
                    msg_v[s, r, pl.ds(dp - _L, _L)] = zero16

        row0 = jnp.minimum(si * _RPT, _ROW_LAST)
        for k in range(_RPT // _BE):
            pltpu.sync_copy(msg_v.at[0], agg.at[pl.ds(row0 + k * _BE, _BE)])
        _rem = _RPT - (_RPT // _BE) * _BE
        if _rem:
            pltpu.sync_copy(msg_v.at[0, pl.ds(0, _rem)],
                            agg.at[pl.ds(row0 + _RPT - _rem, _rem)])

        pltpu.sync_copy(xu_hbm, xu_v)
        pltpu.sync_copy(c_hbm, c_v)
        cvec = c_v[...]
        cs = [cvec[h] for h in range(_H)]

        plsc.subcore_barrier()

        for s in range(2):
            pltpu.async_copy(msg_v.at[s], agg.at[didx_v.at[s]], ssems[s],
                             add=True)

        base_e = w * _EPW

        def stage(k, slot):
            off = base_e + jnp.minimum(k, _NCH - 1) * _BE
            pltpu.async_copy(src_hbm.at[pl.ds(off, _BE)], src_v.at[slot],
                             esems[slot])
            pltpu.async_copy(dst_hbm.at[pl.ds(off, _BE)], dst_v.at[slot],
                             esems[slot])

        def stage_wait(slot):
            for _ in range(2):
                pltpu.make_async_copy(src_hbm.at[pl.ds(base_e, _BE)],
                                      src_v.at[slot], esems[slot]).wait()

        def phase_a(k, slot):
            stage_wait(slot)
            sgs = []
            dgs = []
            for g in range(_BE // _L):
                sg = src_v[slot, pl.ds(g * _L, _L)]
                dg = dst_v[slot, pl.ds(g * _L, _L)]
                dstage_v[slot, pl.ds(g * _L, _L)] = dg
                sidx_v[slot, pl.ds(g * _L, _L)] = sg
                sgs.append(sg)
                dgs.append(dg)
            pltpu.async_copy(xw_hbm.at[sidx_v.at[slot]],
                             rows_v.at[slot], sems[slot])
            for g in range(_BE // _L):
                sg, dg = sgs[g], dgs[g]
                sg4 = sg * _H
                dg4 = dg * _H
                xus = [plsc.load_gather(xu_v, [sg4 + h if h else sg4])
                       for h in range(_H)]
                xud = [plsc.load_gather(xu_v, [dg4 + h if h else dg4])
                       for h in range(_H)]
                ts = [xus[h] - xud[h] + cs[h] for h in range(_H)]
                es = [jnp.exp(ts[h]) for h in range(_H)]
                ssum = (es[0] + es[1]) + (es[2] + es[3])
                validf = jnp.where(sg != dg, 1.0, 0.0).astype(jnp.float32)
                scale = validf / ssum
                for h in range(_H):
                    q_v[slot, h, pl.ds(g * _L, _L)] = es[h] * scale
                q_v[slot, _H, pl.ds(g * _L, _L)] = validf

        def phase_b(slot):
            pltpu.make_async_copy(msg_v.at[slot], agg.at[didx_v.at[slot]],
                                  ssems[slot]).wait()
            for g in range(_BE // _L):
                dg = dstage_v[slot, pl.ds(g * _L, _L)]
                didx_v[slot, pl.ds(g * _L, _L)] = dg
                if dp > _OC:
                    validf = q_v[slot, _H, pl.ds(g * _L, _L)]
                    ridx = lax.iota(jnp.int32, _L) + (g * _L)
                    cidx = jnp.full((_L,), _OC, jnp.int32)
                    plsc.store_scatter(msg_v.at[slot], [ridx, cidx], validf)
            pltpu.make_async_copy(xw_hbm.at[sidx_v.at[slot]],
                                  rows_v.at[slot], sems[slot]).wait()
            for g in range(_BE // _L):
                qv = [q_v[slot, h, pl.ds(g * _L, _L)] for h in range(_H)]
                for lane in range(_L):
                    e = g * _L + lane
                    r0 = rows_v[slot, e, pl.ds(0, _L)]
                    r1 = rows_v[slot, e, pl.ds(_L, _L)]
                    r2 = rows_v[slot, e, pl.ds(2 * _L, _L)]
                    r3 = rows_v[slot, e, pl.ds(3 * _L, _L)]
                    msg_v[slot, e, pl.ds(0, _L)] = (
                        (qv[0][lane] * r0 + qv[1][lane] * r1)
                        + (qv[2][lane] * r2 + qv[3][lane] * r3))
            pltpu.async_copy(msg_v.at[slot], agg.at[didx_v.at[slot]],
                             ssems[slot], add=True)

        stage(0, 0)
        stage(1, 1)
        phase_a(0, 0)
        stage(2, 0)

        def pair_body(k2, c2):
            k = 2 * k2
            phase_a(k + 1, 1)
            stage(k + 3, 1)
            phase_b(0)
            phase_a(k + 2, 0)
            stage(k + 4, 0)
            phase_b(1)
            return c2

        lax.fori_loop(0, (_NCH - 1) // 2, pair_body, 0)
        phase_b(0)
        for s in range(2):
            stage_wait(s)
            pltpu.make_async_copy(msg_v.at[s], agg.at[didx_v.at[s]],
                                  ssems[s]).wait()
        plsc.subcore_barrier()
        pltpu.sync_copy(agg.at[pl.ds(row0, _RPT)],
                        out_hbm.at[pl.ds(ci * _N + row0, _RPT)])

    return edge_kernel


_DPA = 24
_edge_kernel_a = _make_edge_kernel(_DPA)
_edge_kernel_b = _make_edge_kernel(_OC)


def _tc_pre_body(h_ref, W_ref, u_ref, wself_ref, xw_ref, xu_ref, sm_ref):
    h = h_ref[...]
    xw_ref[...] = jnp.dot(h, W_ref[...], preferred_element_type=jnp.float32)
    xu_ref[...] = jnp.dot(h, u_ref[...], preferred_element_type=jnp.float32)
    sm_ref[...] = jnp.dot(h, wself_ref[...], preferred_element_type=jnp.float32)


def _tc_mid1_body(parts_ref, sm_ref, b_ref, dsel_ref, W_ref, u_ref, wself_ref,
                  xw_ref, xu_ref, smn_ref, invdeg_ref):
    s = parts_ref[0:_N, :] + parts_ref[_N:2 * _N, :]
    msg = s[:, 0:_OC]
    deg = jnp.dot(s, dsel_ref[...], preferred_element_type=jnp.float32) + 1.0
    invdeg = 1.0 / deg
    h = jnp.maximum(0.0, (msg + sm_ref[...]) * invdeg + b_ref[...])
    invdeg_ref[...] = invdeg
    xw_ref[...] = jnp.dot(h, W_ref[...], preferred_element_type=jnp.float32)
    xu_ref[...] = jnp.dot(h, u_ref[...], preferred_element_type=jnp.float32)
    smn_ref[...] = jnp.dot(h, wself_ref[...],
                           preferred_element_type=jnp.float32)


def _combine16(parts, sm, b, invdeg):
    s = parts[0:_N, :] + parts[_N:2 * _N, :]
    return jnp.maximum(0.0, (s + sm) * invdeg + b)


def _tc_mid2_body(parts_ref, sm_ref, b_ref, invdeg_ref, W_ref, u_ref,
                  wself_ref, xw_ref, xu_ref, smn_ref):
    h = _combine16(parts_ref[...], sm_ref[...], b_ref[...], invdeg_ref[...])
    xw_ref[...] = jnp.dot(h, W_ref[...], preferred_element_type=jnp.float32)
    xu_ref[...] = jnp.dot(h, u_ref[...], preferred_element_type=jnp.float32)
    smn_ref[...] = jnp.dot(h, wself_ref[...],
                           preferred_element_type=jnp.float32)


def _tc_head_body(parts_ref, sm_ref, b_ref, invdeg_ref, lw1_ref, lb1_ref,
                  lw2_ref, lb2_ref, ow_ref, ob_ref, out_ref, sig_ref):
    h = _combine16(parts_ref[...], sm_ref[...], b_ref[...], invdeg_ref[...])
    h1 = jnp.maximum(0.0, jnp.dot(h, lw1_ref[...],
                                  preferred_element_type=jnp.float32)
                     + lb1_ref[...])
    inter = jnp.dot(h1, lw2_ref[...],
                    preferred_element_type=jnp.float32) + lb2_ref[...]
    h2 = jnp.maximum(0.0, inter)
    out_ref[...] = jax.nn.sigmoid(
        jnp.dot(h2, ow_ref[...], preferred_element_type=jnp.float32)
        + ob_ref[...])
    sig_ref[...] = jax.nn.sigmoid(inter)


def _f32(shape):
    return jax.ShapeDtypeStruct(shape, jnp.float32)


_tc_pre = pl.pallas_call(
    _tc_pre_body,
    out_shape=[_f32((_N, _HOC)), _f32((_N, _H)), _f32((_N, _OC))],
)

_tc_mid1 = pl.pallas_call(
    _tc_mid1_body,
    out_shape=[_f32((_N, _HOC)), _f32((_N, _H)), _f32((_N, _OC)),
               _f32((_N, 1))],
)

_tc_mid2 = pl.pallas_call(
    _tc_mid2_body,
    out_shape=[_f32((_N, _HOC)), _f32((_N, _H)), _f32((_N, _OC))],
)

_tc_head = pl.pallas_call(
    _tc_head_body,
    out_shape=[_f32((_N, 1)), _f32((_N, _H))],
)


def kernel(x, edge_index, W1, u1, c1, b1, W2, u2, c2, b2, W3, u3, c3, b3,
           lw1, lb1, lw2, lb2, ow, ob):
    f32 = jnp.float32
    src = edge_index[0]
    dst = edge_index[1]

    def prep(W, c):
        qs = jax.nn.softmax(c)
        wself = (W.reshape(-1, _H, _OC) * qs[None, :, None]).sum(axis=1)
        cpad = jnp.concatenate([c, jnp.zeros((_L - _H,), f32)])
        return wself, cpad

    wself1, cp1 = prep(W1, c1)
    wself2, cp2 = prep(W2, c2)
    wself3, cp3 = prep(W3, c3)
    dsel = jnp.zeros((_DPA, 1), f32).at[_OC, 0].set(1.0)

    xw, xu, sm = _tc_pre(x, W1, u1, wself1)
    parts = _edge_kernel_a(src, dst, xw, xu.reshape(-1), cp1)
    xw, xu, sm, invdeg = _tc_mid1(parts, sm, b1.reshape(1, _OC), dsel,
                                  W2, u2, wself2)
    parts = _edge_kernel_b(src, dst, xw, xu.reshape(-1), cp2)
    xw, xu, sm = _tc_mid2(parts, sm, b2.reshape(1, _OC), invdeg,
                          W3, u3, wself3)
    parts = _edge_kernel_b(src, dst, xw, xu.reshape(-1), cp3)
    out, sig = _tc_head(parts, sm, b3.reshape(1, _OC), invdeg,
                        lw1, lb1.reshape(1, -1), lw2, lb2.reshape(1, -1),
                        ow, ob.reshape(1, -1))
    return (out, sig)
